# baseline (speedup 1.0000x reference)
import functools

import jax
import jax.numpy as jnp
from jax import lax
from jax.experimental import pallas as pl
from jax.experimental.pallas import tpu as pltpu
from jax.experimental.pallas import tpu_sc as plsc

B = 16384
CTX = 10
NEG = 20
D = 64
V = 1000000
NSCORE = NEG + 1
NC, NS, L = 2, 16, 16
NW = NC * NS
BPW = B // NW
NB = 16
NCHUNK = BPW // NB
NROW = NSCORE + CTX


def _sc_scores(cp, ch, tp, th, np_, nh, in_emb, out_emb):
  mesh = plsc.VectorSubcoreMesh(core_axis_name="c", subcore_axis_name="s")

  @functools.partial(
      pl.kernel,
      out_type=jax.ShapeDtypeStruct((B * NSCORE,), jnp.float32),
      mesh=mesh,
      scratch_types=[
          pltpu.VMEM((BPW * CTX,), jnp.int32),
          pltpu.VMEM((BPW * CTX,), jnp.int32),
          pltpu.VMEM((BPW,), jnp.int32),
          pltpu.VMEM((BPW,), jnp.int32),
          pltpu.VMEM((BPW * NEG,), jnp.int32),
          pltpu.VMEM((BPW * NEG,), jnp.int32),
          pltpu.VMEM((CTX * NB, 2 * D), jnp.float32),
          pltpu.VMEM((NB, 2 * D), jnp.float32),
          pltpu.VMEM((NEG * NB, 2 * D), jnp.float32),
          pltpu.VMEM((NROW * L,), jnp.int32),
          pltpu.VMEM((BPW * NSCORE,), jnp.float32),
          pltpu.SemaphoreType.DMA,
          pltpu.SemaphoreType.DMA,
      ],
      compiler_params=pltpu.CompilerParams(
          needs_layout_passes=False, use_tc_tiling_on_sc=False),
  )
  def k(cp_hbm, ch_hbm, tp_hbm, th_hbm, np_hbm, nh_hbm, ine_hbm, oute_hbm,
        out_hbm, cp_s, ch_s, tp_s, th_s, np_s, nh_s,
        ctx_rows, tgt_rows, neg_rows, rq_s, out_all, sem_i, sem_r):
    wid = lax.axis_index("s") * NC + lax.axis_index("c")
    iota = lax.iota(jnp.int32, L)
    rows10 = iota * CTX
    rows20 = iota * NEG

    ine = ine_hbm
    oute = oute_hbm

    cps = [
        pltpu.async_copy(cp_hbm.at[pl.ds(wid * BPW * CTX, BPW * CTX)],
                         cp_s, sem_i),
        pltpu.async_copy(ch_hbm.at[pl.ds(wid * BPW * CTX, BPW * CTX)],
                         ch_s, sem_i),
        pltpu.async_copy(tp_hbm.at[pl.ds(wid * BPW, BPW)], tp_s, sem_i),
        pltpu.async_copy(th_hbm.at[pl.ds(wid * BPW, BPW)], th_s, sem_i),
        pltpu.async_copy(np_hbm.at[pl.ds(wid * BPW * NEG, BPW * NEG)],
                         np_s, sem_i),
        pltpu.async_copy(nh_hbm.at[pl.ds(wid * BPW * NEG, BPW * NEG)],
                         nh_s, sem_i),
    ]
    for c in cps:
      c.wait()

    def chunk(g, carry):
      co = g * CTX * NB
      no = g * NEG * NB
      gs = [
          pltpu.async_copy(ine.at[cp_s.at[pl.ds(co, 128)]],
                           ctx_rows.at[pl.ds(0, 128), :], sem_r),
          pltpu.async_copy(ine.at[cp_s.at[pl.ds(co + 128, 32)]],
                           ctx_rows.at[pl.ds(128, 32), :], sem_r),
          pltpu.async_copy(oute.at[tp_s.at[pl.ds(g * NB, NB)]],
                           tgt_rows, sem_r),
          pltpu.async_copy(oute.at[np_s.at[pl.ds(no, 128)]],
                           neg_rows.at[pl.ds(0, 128), :], sem_r),
          pltpu.async_copy(oute.at[np_s.at[pl.ds(no + 128, 128)]],
                           neg_rows.at[pl.ds(128, 128), :], sem_r),
          pltpu.async_copy(oute.at[np_s.at[pl.ds(no + 256, 64)]],
                           neg_rows.at[pl.ds(256, 64), :], sem_r),
      ]

      for j in range(CTX):
        h = plsc.load_gather(ch_s, [co + rows10 + j])
        rq_s[pl.ds(j * L, L)] = h * D
      ht = plsc.load_gather(th_s, [g * NB + iota])
      rq_s[pl.ds(CTX * L, L)] = ht * D
      for n in range(NEG):
        h = plsc.load_gather(nh_s, [no + rows20 + n])
        rq_s[pl.ds((CTX + 1 + n) * L, L)] = h * D

      for gd in gs:
        gd.wait()

      def dstep(d, acc):
        dv = (iota + d) & (D - 1)
        c = plsc.load_gather(ctx_rows, [rows10, rq_s[pl.ds(0, L)] + dv])
        for j in range(1, CTX):
          c = c + plsc.load_gather(ctx_rows,
                                   [rows10 + j, rq_s[pl.ds(j * L, L)] + dv])
        t = plsc.load_gather(tgt_rows, [iota, rq_s[pl.ds(CTX * L, L)] + dv])
        pos = acc[0] + c * t
        negs = [
            acc[1 + n]
            + c * plsc.load_gather(
                neg_rows,
                [rows20 + n, rq_s[pl.ds((CTX + 1 + n) * L, L)] + dv])
            for n in range(NEG)
        ]
        return (pos, *negs)

      zero = jnp.zeros((L,), jnp.float32)
      acc = lax.fori_loop(0, D, dstep, (zero,) * NSCORE)
      scale = jnp.float32(1.0 / CTX)
      oidx = (g * NB + iota) * NSCORE
      plsc.store_scatter(out_all, [oidx], acc[0] * scale)
      for n in range(NEG):
        plsc.store_scatter(out_all, [oidx + (1 + n)], acc[1 + n] * (-scale))
      return carry

    lax.fori_loop(0, NCHUNK, chunk, 0)

    pltpu.sync_copy(
        out_all, out_hbm.at[pl.ds(wid * BPW * NSCORE, BPW * NSCORE)])

  return k(cp, ch, tp, th, np_, nh, in_emb, out_emb)


def _tc_loss(scores2d):
  def body(x_ref, o_ref):
    ls = jax.nn.log_sigmoid(x_ref[...])
    o_ref[0, 0] = -jnp.sum(ls) / jnp.float32(B)

  return pl.pallas_call(
      body,
      out_shape=jax.ShapeDtypeStruct((1, 1), jnp.float32),
      out_specs=pl.BlockSpec(memory_space=pltpu.SMEM),
  )(scores2d)


def kernel(context_words, target_words, negative_words, input_embeddings,
           output_embeddings):
  ctx = context_words.reshape(-1).astype(jnp.int32)
  neg = negative_words.reshape(-1).astype(jnp.int32)
  tgt = target_words.astype(jnp.int32)
  in2 = input_embeddings.reshape(V // 2, 2 * D)
  out2 = output_embeddings.reshape(V // 2, 2 * D)
  scores = _sc_scores(ctx >> 1, ctx & 1, tgt >> 1, tgt & 1, neg >> 1,
                      neg & 1, in2, out2)
  loss = _tc_loss(scores.reshape(B * NSCORE // 128, 128))
  return loss[0, 0]


# speedup vs baseline: 1.0868x; 1.0868x over previous
import functools

import jax
import jax.numpy as jnp
from jax import lax
from jax.experimental import pallas as pl
from jax.experimental.pallas import tpu as pltpu
from jax.experimental.pallas import tpu_sc as plsc

B = 16384
CTX = 10
NEG = 20
D = 64
NSCORE = NEG + 1
NC, NS, L = 2, 16, 16
NW = NC * NS
BPW = B // NW
NB = 16
NCHUNK = BPW // NB
NBUF = 2


def _sc_scores(ctx_flat, tgt, neg_flat, in_emb, out_emb):
  mesh = plsc.VectorSubcoreMesh(core_axis_name="c", subcore_axis_name="s")

  @functools.partial(
      pl.kernel,
      out_type=jax.ShapeDtypeStruct((B * NSCORE,), jnp.float32),
      mesh=mesh,
      scratch_types=[
          pltpu.VMEM((BPW * CTX,), jnp.int32),
          pltpu.VMEM((BPW,), jnp.int32),
          pltpu.VMEM((BPW * NEG,), jnp.int32),
          pltpu.VMEM((NBUF, CTX * NB, D), jnp.float32),
          pltpu.VMEM((NBUF, NB, D), jnp.float32),
          pltpu.VMEM((NBUF, NEG * NB, D), jnp.float32),
          pltpu.VMEM((BPW * NSCORE,), jnp.float32),
          pltpu.SemaphoreType.DMA,
          pltpu.SemaphoreType.DMA,
          pltpu.SemaphoreType.DMA,
      ],
      compiler_params=pltpu.CompilerParams(
          needs_layout_passes=False, use_tc_tiling_on_sc=False),
  )
  def k(ctx_hbm, tgt_hbm, neg_hbm, ine_hbm, oute_hbm, out_hbm,
        ctx_idx, tgt_idx, neg_idx, ctx_rows, tgt_rows, neg_rows,
        out_all, sem_i, sem_r0, sem_r1):
    wid = lax.axis_index("s") * NC + lax.axis_index("c")
    iota = lax.iota(jnp.int32, L)
    rows10 = iota * CTX
    rows20 = iota * NEG
    sems = (sem_r0, sem_r1)

    def fire(g, b):
      sem = sems[b]
      co = g * CTX * NB
      no = g * NEG * NB
      pltpu.async_copy(ine_hbm.at[ctx_idx.at[pl.ds(co, 128)]],
                       ctx_rows.at[b, pl.ds(0, 128), :], sem)
      pltpu.async_copy(ine_hbm.at[ctx_idx.at[pl.ds(co + 128, 32)]],
                       ctx_rows.at[b, pl.ds(128, 32), :], sem)
      pltpu.async_copy(oute_hbm.at[tgt_idx.at[pl.ds(g * NB, NB)]],
                       tgt_rows.at[b], sem)
      pltpu.async_copy(oute_hbm.at[neg_idx.at[pl.ds(no, 128)]],
                       neg_rows.at[b, pl.ds(0, 128), :], sem)
      pltpu.async_copy(oute_hbm.at[neg_idx.at[pl.ds(no + 128, 128)]],
                       neg_rows.at[b, pl.ds(128, 128), :], sem)
      pltpu.async_copy(oute_hbm.at[neg_idx.at[pl.ds(no + 256, 64)]],
                       neg_rows.at[b, pl.ds(256, 64), :], sem)

    def drain(b):
      sem = sems[b]
      pltpu.make_async_copy(ine_hbm.at[ctx_idx.at[pl.ds(0, 128)]],
                            ctx_rows.at[b, pl.ds(0, 128), :], sem).wait()
      pltpu.make_async_copy(ine_hbm.at[ctx_idx.at[pl.ds(0, 32)]],
                            ctx_rows.at[b, pl.ds(128, 32), :], sem).wait()
      pltpu.make_async_copy(oute_hbm.at[tgt_idx.at[pl.ds(0, NB)]],
                            tgt_rows.at[b], sem).wait()
      pltpu.make_async_copy(oute_hbm.at[neg_idx.at[pl.ds(0, 128)]],
                            neg_rows.at[b, pl.ds(0, 128), :], sem).wait()
      pltpu.make_async_copy(oute_hbm.at[neg_idx.at[pl.ds(0, 128)]],
                            neg_rows.at[b, pl.ds(128, 128), :], sem).wait()
      pltpu.make_async_copy(oute_hbm.at[neg_idx.at[pl.ds(0, 64)]],
                            neg_rows.at[b, pl.ds(256, 64), :], sem).wait()

    def compute(g, b):
      def dstep(d, acc):
        dv = (iota + d) & (D - 1)
        c = plsc.load_gather(ctx_rows.at[b], [rows10, dv])
        for j in range(1, CTX):
          c = c + plsc.load_gather(ctx_rows.at[b], [rows10 + j, dv])
        t = plsc.load_gather(tgt_rows.at[b], [iota, dv])
        pos = acc[0] + c * t
        negs = [
            acc[1 + n] + c * plsc.load_gather(neg_rows.at[b], [rows20 + n, dv])
            for n in range(NEG)
        ]
        return (pos, *negs)

      zero = jnp.zeros((L,), jnp.float32)
      acc = lax.fori_loop(0, D, dstep, (zero,) * NSCORE)
      scale = jnp.float32(1.0 / CTX)
      oidx = (g * NB + iota) * NSCORE
      plsc.store_scatter(out_all, [oidx], acc[0] * scale)
      for n in range(NEG):
        plsc.store_scatter(out_all, [oidx + (1 + n)], acc[1 + n] * (-scale))

    ci = pltpu.async_copy(
        ctx_hbm.at[pl.ds(wid * BPW * CTX, BPW * CTX)], ctx_idx, sem_i)
    ti = pltpu.async_copy(tgt_hbm.at[pl.ds(wid * BPW, BPW)], tgt_idx, sem_i)
    ni = pltpu.async_copy(
        neg_hbm.at[pl.ds(wid * BPW * NEG, BPW * NEG)], neg_idx, sem_i)
    ci.wait()
    ti.wait()
    ni.wait()

    for b in range(NBUF):
      fire(b, b)

    def pair(p, carry):
      g = p * NBUF
      for b in range(NBUF):
        drain(b)
        compute(g + b, b)
        @pl.when(g + b + NBUF < NCHUNK)
        def _():
          fire(g + b + NBUF, b)
      return carry

    lax.fori_loop(0, NCHUNK // NBUF, pair, 0)

    pltpu.sync_copy(
        out_all, out_hbm.at[pl.ds(wid * BPW * NSCORE, BPW * NSCORE)])

  return k(ctx_flat, tgt, neg_flat, in_emb, out_emb)


def _tc_loss(scores2d):
  def body(x_ref, o_ref):
    ls = jax.nn.log_sigmoid(x_ref[...])
    o_ref[0, 0] = -jnp.sum(ls) / jnp.float32(B)

  return pl.pallas_call(
      body,
      out_shape=jax.ShapeDtypeStruct((1, 1), jnp.float32),
      out_specs=pl.BlockSpec(memory_space=pltpu.SMEM),
  )(scores2d)


def kernel(context_words, target_words, negative_words, input_embeddings,
           output_embeddings):
  ctx_flat = context_words.reshape(-1).astype(jnp.int32)
  neg_flat = negative_words.reshape(-1).astype(jnp.int32)
  tgt = target_words.astype(jnp.int32)
  scores = _sc_scores(ctx_flat, tgt, neg_flat, input_embeddings,
                      output_embeddings)
  loss = _tc_loss(scores.reshape(B * NSCORE // 128, 128))
  return loss[0, 0]
